# baseline (speedup 1.0000x reference)
import functools

import jax
import jax.numpy as jnp
from jax import lax
from jax.experimental import pallas as pl
from jax.experimental.pallas import tpu as pltpu
from jax.experimental.pallas import tpu_sc as plsc

N = 100000
D = 32
H = 16
E = 1600000
NS = 16
RPT = 784
ROWS = RPT * NS
EPAD = ROWS * 128
RB = 8
NBLK = RPT // RB
RBA = 4
NBLKA = RPT // RBA
NACC = 100352
ZSTRIPE = NACC // NS
NP = NACC // 8
BP = NP // 8
GRID = NP // BP

_mesh = plsc.VectorSubcoreMesh(core_axis_name="c", subcore_axis_name="s")


@functools.partial(
    pl.kernel,
    out_type=[jax.ShapeDtypeStruct((NACC, H), jnp.float32)] * 4,
    mesh=_mesh,
    compiler_params=pltpu.CompilerParams(use_tc_tiling_on_sc=False),
    scratch_types=[
        pltpu.VMEM_SHARED((NACC, H), jnp.float32),
        pltpu.VMEM((RBA, 128), jnp.int32),
        pltpu.VMEM((RBA, 128), jnp.int32),
        pltpu.VMEM((RBA, 128), jnp.int32),
        pltpu.VMEM((RBA, 128), jnp.int32),
        pltpu.VMEM((RBA * 128, H), jnp.float32),
        pltpu.VMEM((RBA * 128, H), jnp.float32),
        pltpu.SemaphoreType.DMA,
        pltpu.SemaphoreType.DMA,
        pltpu.SemaphoreType.DMA,
        pltpu.SemaphoreType.DMA,
    ],
)
def _sc_agg(tA_r, tB_r, tA_b, tB_b, srcR, dstR, srcB, dstB, zeros,
            oAi, oBi, oAu, oBu, accum,
            sidx0, sidx1, didx0, didx1, rows0, rows1,
            semg0, semg1, sems0, sems1):
    c = lax.axis_index("c")
    s = lax.axis_index("s")
    sidx = (sidx0, sidx1)
    didx = (didx0, didx1)
    rows = (rows0, rows1)
    semg = (semg0, semg1)
    sems = (sems0, sems1)

    def do_rel(table, src2, dst2, out):
        pltpu.sync_copy(zeros, accum.at[pl.ds(s * ZSTRIPE, ZSTRIPE)])
        plsc.subcore_barrier()
        base = s * RPT

        def load_idx(st, r0):
            pltpu.sync_copy(src2.at[pl.ds(r0, RBA)], sidx[st])
            pltpu.sync_copy(dst2.at[pl.ds(r0, RBA)], didx[st])

        def fire_gathers(st):
            for j in range(RBA):
                pltpu.async_copy(table.at[sidx[st].at[j]],
                                 rows[st].at[pl.ds(j * 128, 128)], semg[st])

        def drain_gathers(st):
            for j in range(RBA):
                pltpu.make_async_copy(table.at[sidx[st].at[j]],
                                      rows[st].at[pl.ds(j * 128, 128)],
                                      semg[st]).wait()

        def fire_scatters(st):
            for j in range(RBA):
                pltpu.async_copy(rows[st].at[pl.ds(j * 128, 128)],
                                 accum.at[didx[st].at[j]], sems[st], add=True)

        def drain_scatters(st):
            for j in range(RBA):
                pltpu.make_async_copy(rows[st].at[pl.ds(j * 128, 128)],
                                      accum.at[didx[st].at[j]],
                                      sems[st]).wait()

        load_idx(0, base)
        fire_gathers(0)

        def pair(k, carry):
            for st in range(2):
                b = k * 2 + st
                nst = 1 - st
                drain_gathers(st)
                fire_scatters(st)

                @pl.when(b > 0)
                def _():
                    drain_scatters(nst)

                @pl.when(b + 1 < NBLKA)
                def _():
                    load_idx(nst, base + (b + 1) * RBA)
                    fire_gathers(nst)
            return carry

        lax.fori_loop(0, NBLKA // 2, pair, 0)
        drain_scatters((NBLKA - 1) % 2)
        plsc.subcore_barrier()
        pltpu.sync_copy(accum.at[pl.ds(s * ZSTRIPE, ZSTRIPE)],
                        out.at[pl.ds(s * ZSTRIPE, ZSTRIPE)])
        plsc.subcore_barrier()

    @pl.when(c == 0)
    def _():
        do_rel(tA_r, srcR, dstR, oAi)
        do_rel(tA_b, srcB, dstB, oAu)

    @pl.when(c == 1)
    def _():
        do_rel(tB_r, srcR, dstR, oBi)
        do_rel(tB_b, srcB, dstB, oBu)


@functools.partial(
    pl.kernel,
    out_type=[jax.ShapeDtypeStruct((NACC, H), jnp.float32)] * 2,
    mesh=_mesh,
    compiler_params=pltpu.CompilerParams(use_tc_tiling_on_sc=False),
    scratch_types=[
        pltpu.VMEM_SHARED((NACC, H), jnp.float32),
        pltpu.VMEM((RB, 128), jnp.int32),
        pltpu.VMEM((128, H), jnp.float32),
        pltpu.SemaphoreType.DMA,
    ],
)
def _sc_deg(dstR, dstB, ones, zeros, oI, oU, accum, didx, ones_v, sem):
    c = lax.axis_index("c")
    s = lax.axis_index("s")
    pltpu.sync_copy(ones, ones_v)

    def do_rel(dst2, out):
        pltpu.sync_copy(zeros, accum.at[pl.ds(s * ZSTRIPE, ZSTRIPE)])
        plsc.subcore_barrier()
        base = s * RPT

        def blk(b, carry):
            r0 = base + b * RB
            pltpu.sync_copy(dst2.at[pl.ds(r0, RB)], didx)
            scs = [pltpu.async_copy(ones_v, accum.at[didx.at[j]], sem,
                                    add=True)
                   for j in range(RB)]
            for cp in scs:
                cp.wait()
            return carry

        lax.fori_loop(0, NBLK, blk, 0)
        plsc.subcore_barrier()
        pltpu.sync_copy(accum.at[pl.ds(s * ZSTRIPE, ZSTRIPE)],
                        out.at[pl.ds(s * ZSTRIPE, ZSTRIPE)])

    @pl.when(c == 0)
    def _():
        do_rel(dstR, oI)

    @pl.when(c == 1)
    def _():
        do_rel(dstB, oU)


def _rows(w):
    return pl.BlockSpec((BP, w), lambda i: (i, 0))


def _const(r, w):
    return pl.BlockSpec((r, w), lambda i: (0, 0))


_f32 = functools.partial(jnp.dot, preferred_element_type=jnp.float32)


def _tcA_body(xu, xi, wrA, wrB, wbA, wbB, yAr, yBr, yAb, yBb):
    yAr[...] = _f32(xu[...], wrA[...])
    yBr[...] = _f32(xu[...], wrB[...])
    yAb[...] = _f32(xi[...], wbA[...])
    yBb[...] = _f32(xi[...], wbB[...])


_tcA = pl.pallas_call(
    _tcA_body,
    grid=(GRID,),
    in_specs=[_rows(256), _rows(256)] + [_const(256, 128)] * 4,
    out_specs=[_rows(128)] * 4,
    out_shape=[jax.ShapeDtypeStruct((NP, 128), jnp.float32)] * 4,
)


def _mix(aA, aB, dg, eA, eB):
    inv = 1.0 / jnp.maximum(dg, 1.0)
    return _f32(aA * inv, eA) + _f32(aB * inv, eB)


def _tcB_body(xu, aAu, aBu, dgu, wsu, bu, w2rA, w2rB,
              xi, aAi, aBi, dgi, wsi, bi, w2bA, w2bB, eA, eB,
              hu, t2Ar, t2Br, hi, t2Ab, t2Bb):
    h_u = jnp.maximum(_f32(xu[...], wsu[...])
                      + _mix(aAu[...], aBu[...], dgu[...], eA[...], eB[...])
                      + bu[...], 0.0)
    hu[...] = h_u
    t2Ar[...] = _f32(h_u, w2rA[...])
    t2Br[...] = _f32(h_u, w2rB[...])
    h_i = jnp.maximum(_f32(xi[...], wsi[...])
                      + _mix(aAi[...], aBi[...], dgi[...], eA[...], eB[...])
                      + bi[...], 0.0)
    hi[...] = h_i
    t2Ab[...] = _f32(h_i, w2bA[...])
    t2Bb[...] = _f32(h_i, w2bB[...])


_tcB = pl.pallas_call(
    _tcB_body,
    grid=(GRID,),
    in_specs=[_rows(256), _rows(128), _rows(128), _rows(128),
              _const(256, 256), _const(1, 256), _const(256, 128), _const(256, 128),
              _rows(256), _rows(128), _rows(128), _rows(128),
              _const(256, 256), _const(1, 256), _const(256, 128), _const(256, 128),
              _const(128, 256), _const(128, 256)],
    out_specs=[_rows(256), _rows(128), _rows(128),
               _rows(256), _rows(128), _rows(128)],
    out_shape=[jax.ShapeDtypeStruct((NP, 256), jnp.float32),
               jax.ShapeDtypeStruct((NP, 128), jnp.float32),
               jax.ShapeDtypeStruct((NP, 128), jnp.float32),
               jax.ShapeDtypeStruct((NP, 256), jnp.float32),
               jax.ShapeDtypeStruct((NP, 128), jnp.float32),
               jax.ShapeDtypeStruct((NP, 128), jnp.float32)],
)


def _tcC_body(hu, aAu, aBu, dgu, wsu, bu,
              hi, aAi, aBi, dgi, wsi, bi, eA, eB,
              ou, oi):
    ou[...] = (_f32(hu[...], wsu[...])
               + _mix(aAu[...], aBu[...], dgu[...], eA[...], eB[...])
               + bu[...])
    oi[...] = (_f32(hi[...], wsi[...])
               + _mix(aAi[...], aBi[...], dgi[...], eA[...], eB[...])
               + bi[...])


_tcC = pl.pallas_call(
    _tcC_body,
    grid=(GRID,),
    in_specs=[_rows(256), _rows(128), _rows(128), _rows(128),
              _const(256, 256), _const(1, 256),
              _rows(256), _rows(128), _rows(128), _rows(128),
              _const(256, 256), _const(1, 256),
              _const(128, 256), _const(128, 256)],
    out_specs=[_rows(256), _rows(256)],
    out_shape=[jax.ShapeDtypeStruct((NP, 256), jnp.float32),
               jax.ShapeDtypeStruct((NP, 256), jnp.float32)],
)


def _pad_edges(src, dst):
    padn = EPAD - E
    pad_src = (jnp.arange(padn, dtype=jnp.int32) * 97) % N
    pad_dst = N + (jnp.arange(padn, dtype=jnp.int32) % (NACC - N))
    src2 = jnp.concatenate([src.astype(jnp.int32), pad_src]).reshape(ROWS, 128)
    dst2 = jnp.concatenate([dst.astype(jnp.int32), pad_dst]).reshape(ROWS, 128)
    return src2, dst2


def _pack(x):
    return jnp.pad(x, ((0, NACC - N), (0, 0))).reshape(NP, 8 * x.shape[1])


def _bd(W):
    return jnp.kron(jnp.eye(8, dtype=W.dtype), W)


def _as_table(p):
    return p.reshape(NACC, H)


def _as_packed(t):
    return t.reshape(NP, 128)


def kernel(x_user, x_item, edge_index_rates, edge_index_rated_by,
           W1_rates_self, W1_rates_neigh, W1_rb_self, W1_rb_neigh,
           W2_rates_self, W2_rates_neigh, W2_rb_self, W2_rb_neigh,
           b1_rates, b1_rb, b2_rates, b2_rb):
    srcR2, dstR2 = _pad_edges(edge_index_rates[0], edge_index_rates[1])
    srcB2, dstB2 = _pad_edges(edge_index_rated_by[0], edge_index_rated_by[1])
    zeros = jnp.zeros((ZSTRIPE, H), jnp.float32)
    ones = jnp.ones((128, H), jnp.float32)
    eye8 = jnp.eye(8, dtype=jnp.float32)
    selA = jnp.kron(eye8, jnp.eye(H, D, dtype=jnp.float32))
    selB = jnp.kron(eye8, jnp.eye(H, D, k=H, dtype=jnp.float32))

    xu_p = _pack(x_user)
    xi_p = _pack(x_item)

    degI, degU = _sc_deg(dstR2, dstB2, ones, zeros)
    degI_p, degU_p = _as_packed(degI), _as_packed(degU)

    yAr, yBr, yAb, yBb = _tcA(xu_p, xi_p,
                              _bd(W1_rates_neigh[:, :H]), _bd(W1_rates_neigh[:, H:]),
                              _bd(W1_rb_neigh[:, :H]), _bd(W1_rb_neigh[:, H:]))
    aAi, aBi, aAu, aBu = _sc_agg(_as_table(yAr), _as_table(yBr),
                                 _as_table(yAb), _as_table(yBb),
                                 srcR2, dstR2, srcB2, dstB2, zeros)
    hu, t2Ar, t2Br, hi, t2Ab, t2Bb = _tcB(
        xu_p, _as_packed(aAu), _as_packed(aBu), degU_p,
        _bd(W1_rb_self), jnp.tile(b1_rb, 8).reshape(1, 256),
        _bd(W2_rates_neigh[:, :H]), _bd(W2_rates_neigh[:, H:]),
        xi_p, _as_packed(aAi), _as_packed(aBi), degI_p,
        _bd(W1_rates_self), jnp.tile(b1_rates, 8).reshape(1, 256),
        _bd(W2_rb_neigh[:, :H]), _bd(W2_rb_neigh[:, H:]),
        selA, selB)
    a2Ai, a2Bi, a2Au, a2Bu = _sc_agg(_as_table(t2Ar), _as_table(t2Br),
                                     _as_table(t2Ab), _as_table(t2Bb),
                                     srcR2, dstR2, srcB2, dstB2, zeros)
    hu2, hi2 = _tcC(hu, _as_packed(a2Au), _as_packed(a2Bu), degU_p,
                    _bd(W2_rb_self), jnp.tile(b2_rb, 8).reshape(1, 256),
                    hi, _as_packed(a2Ai), _as_packed(a2Bi), degI_p,
                    _bd(W2_rates_self), jnp.tile(b2_rates, 8).reshape(1, 256),
                    selA, selB)
    out_u = hu2.reshape(NACC, D)[:N]
    out_i = hi2.reshape(NACC, D)[:N]
    return (out_u, out_i)


# speedup vs baseline: 1.3133x; 1.3133x over previous
import functools

import jax
import jax.numpy as jnp
from jax import lax
from jax.experimental import pallas as pl
from jax.experimental.pallas import tpu as pltpu
from jax.experimental.pallas import tpu_sc as plsc

N = 100000
D = 32
H = 16
E = 1600000
NS = 16
RPT = 784
ROWS = RPT * NS
EPAD = ROWS * 128
EPT = RPT * 128
BE = 1024
NBLK = EPT // BE
NACC = 100352
ZSTRIPE = NACC // NS
NP = NACC // 8
BP = NP // 8
GRID = NP // BP

_mesh = plsc.VectorSubcoreMesh(core_axis_name="c", subcore_axis_name="s")


@functools.partial(
    pl.kernel,
    out_type=[jax.ShapeDtypeStruct((NACC, H), jnp.float32)] * 4,
    mesh=_mesh,
    compiler_params=pltpu.CompilerParams(use_tc_tiling_on_sc=False),
    scratch_types=[
        pltpu.VMEM_SHARED((NACC, H), jnp.float32),
        pltpu.VMEM((BE,), jnp.int32),
        pltpu.VMEM((BE,), jnp.int32),
        pltpu.VMEM((BE, H), jnp.float32),
        pltpu.SemaphoreType.DMA,
    ],
)
def _sc_agg(tA_r, tB_r, tA_b, tB_b, srcR, dstR, srcB, dstB, zeros,
            oAi, oBi, oAu, oBu, accum, sidx, didx, rows, sem):
    c = lax.axis_index("c")
    s = lax.axis_index("s")

    def do_rel(table, src1, dst1, out):
        pltpu.sync_copy(zeros, accum.at[pl.ds(s * ZSTRIPE, ZSTRIPE)])
        plsc.subcore_barrier()
        base = s * EPT

        def blk(b, carry):
            e0 = base + b * BE
            ics = [pltpu.async_copy(src1.at[pl.ds(e0, BE)], sidx, sem),
                   pltpu.async_copy(dst1.at[pl.ds(e0, BE)], didx, sem)]
            for cp in ics:
                cp.wait()
            pltpu.async_copy(table.at[sidx], rows, sem).wait()
            pltpu.async_copy(rows, accum.at[didx], sem, add=True).wait()
            return carry

        lax.fori_loop(0, NBLK, blk, 0)
        plsc.subcore_barrier()
        pltpu.sync_copy(accum.at[pl.ds(s * ZSTRIPE, ZSTRIPE)],
                        out.at[pl.ds(s * ZSTRIPE, ZSTRIPE)])
        plsc.subcore_barrier()

    @pl.when(c == 0)
    def _():
        do_rel(tA_r, srcR, dstR, oAi)
        do_rel(tA_b, srcB, dstB, oAu)

    @pl.when(c == 1)
    def _():
        do_rel(tB_r, srcR, dstR, oBi)
        do_rel(tB_b, srcB, dstB, oBu)


@functools.partial(
    pl.kernel,
    out_type=[jax.ShapeDtypeStruct((NACC, H), jnp.float32)] * 2,
    mesh=_mesh,
    compiler_params=pltpu.CompilerParams(use_tc_tiling_on_sc=False),
    scratch_types=[
        pltpu.VMEM_SHARED((NACC, H), jnp.float32),
        pltpu.VMEM((BE,), jnp.int32),
        pltpu.VMEM((BE, H), jnp.float32),
        pltpu.SemaphoreType.DMA,
    ],
)
def _sc_deg(dstR, dstB, ones, zeros, oI, oU, accum, didx, ones_v, sem):
    c = lax.axis_index("c")
    s = lax.axis_index("s")
    pltpu.sync_copy(ones, ones_v)

    def do_rel(dst1, out):
        pltpu.sync_copy(zeros, accum.at[pl.ds(s * ZSTRIPE, ZSTRIPE)])
        plsc.subcore_barrier()
        base = s * EPT

        def blk(b, carry):
            e0 = base + b * BE
            pltpu.sync_copy(dst1.at[pl.ds(e0, BE)], didx)
            pltpu.async_copy(ones_v, accum.at[didx], sem, add=True).wait()
            return carry

        lax.fori_loop(0, NBLK, blk, 0)
        plsc.subcore_barrier()
        pltpu.sync_copy(accum.at[pl.ds(s * ZSTRIPE, ZSTRIPE)],
                        out.at[pl.ds(s * ZSTRIPE, ZSTRIPE)])

    @pl.when(c == 0)
    def _():
        do_rel(dstR, oI)

    @pl.when(c == 1)
    def _():
        do_rel(dstB, oU)


def _rows(w):
    return pl.BlockSpec((BP, w), lambda i: (i, 0))


def _const(r, w):
    return pl.BlockSpec((r, w), lambda i: (0, 0))


_f32 = functools.partial(jnp.dot, preferred_element_type=jnp.float32)


def _tcA_body(xu, xi, wrA, wrB, wbA, wbB, yAr, yBr, yAb, yBb):
    yAr[...] = _f32(xu[...], wrA[...])
    yBr[...] = _f32(xu[...], wrB[...])
    yAb[...] = _f32(xi[...], wbA[...])
    yBb[...] = _f32(xi[...], wbB[...])


_tcA = pl.pallas_call(
    _tcA_body,
    grid=(GRID,),
    in_specs=[_rows(256), _rows(256)] + [_const(256, 128)] * 4,
    out_specs=[_rows(128)] * 4,
    out_shape=[jax.ShapeDtypeStruct((NP, 128), jnp.float32)] * 4,
)


def _mix(aA, aB, dg, eA, eB):
    inv = 1.0 / jnp.maximum(dg, 1.0)
    return _f32(aA * inv, eA) + _f32(aB * inv, eB)


def _tcB_body(xu, aAu, aBu, dgu, wsu, bu, w2rA, w2rB,
              xi, aAi, aBi, dgi, wsi, bi, w2bA, w2bB, eA, eB,
              hu, t2Ar, t2Br, hi, t2Ab, t2Bb):
    h_u = jnp.maximum(_f32(xu[...], wsu[...])
                      + _mix(aAu[...], aBu[...], dgu[...], eA[...], eB[...])
                      + bu[...], 0.0)
    hu[...] = h_u
    t2Ar[...] = _f32(h_u, w2rA[...])
    t2Br[...] = _f32(h_u, w2rB[...])
    h_i = jnp.maximum(_f32(xi[...], wsi[...])
                      + _mix(aAi[...], aBi[...], dgi[...], eA[...], eB[...])
                      + bi[...], 0.0)
    hi[...] = h_i
    t2Ab[...] = _f32(h_i, w2bA[...])
    t2Bb[...] = _f32(h_i, w2bB[...])


_tcB = pl.pallas_call(
    _tcB_body,
    grid=(GRID,),
    in_specs=[_rows(256), _rows(128), _rows(128), _rows(128),
              _const(256, 256), _const(1, 256), _const(256, 128), _const(256, 128),
              _rows(256), _rows(128), _rows(128), _rows(128),
              _const(256, 256), _const(1, 256), _const(256, 128), _const(256, 128),
              _const(128, 256), _const(128, 256)],
    out_specs=[_rows(256), _rows(128), _rows(128),
               _rows(256), _rows(128), _rows(128)],
    out_shape=[jax.ShapeDtypeStruct((NP, 256), jnp.float32),
               jax.ShapeDtypeStruct((NP, 128), jnp.float32),
               jax.ShapeDtypeStruct((NP, 128), jnp.float32),
               jax.ShapeDtypeStruct((NP, 256), jnp.float32),
               jax.ShapeDtypeStruct((NP, 128), jnp.float32),
               jax.ShapeDtypeStruct((NP, 128), jnp.float32)],
)


def _tcC_body(hu, aAu, aBu, dgu, wsu, bu,
              hi, aAi, aBi, dgi, wsi, bi, eA, eB,
              ou, oi):
    ou[...] = (_f32(hu[...], wsu[...])
               + _mix(aAu[...], aBu[...], dgu[...], eA[...], eB[...])
               + bu[...])
    oi[...] = (_f32(hi[...], wsi[...])
               + _mix(aAi[...], aBi[...], dgi[...], eA[...], eB[...])
               + bi[...])


_tcC = pl.pallas_call(
    _tcC_body,
    grid=(GRID,),
    in_specs=[_rows(256), _rows(128), _rows(128), _rows(128),
              _const(256, 256), _const(1, 256),
              _rows(256), _rows(128), _rows(128), _rows(128),
              _const(256, 256), _const(1, 256),
              _const(128, 256), _const(128, 256)],
    out_specs=[_rows(256), _rows(256)],
    out_shape=[jax.ShapeDtypeStruct((NP, 256), jnp.float32),
               jax.ShapeDtypeStruct((NP, 256), jnp.float32)],
)


def _pad_edges(src, dst):
    padn = EPAD - E
    pad_src = (jnp.arange(padn, dtype=jnp.int32) * 97) % N
    pad_dst = N + (jnp.arange(padn, dtype=jnp.int32) % (NACC - N))
    src1 = jnp.concatenate([src.astype(jnp.int32), pad_src])
    dst1 = jnp.concatenate([dst.astype(jnp.int32), pad_dst])
    return src1, dst1


def _pack(x):
    return jnp.pad(x, ((0, NACC - N), (0, 0))).reshape(NP, 8 * x.shape[1])


def _bd(W):
    return jnp.kron(jnp.eye(8, dtype=W.dtype), W)


def _as_table(p):
    return p.reshape(NACC, H)


def _as_packed(t):
    return t.reshape(NP, 128)


def kernel(x_user, x_item, edge_index_rates, edge_index_rated_by,
           W1_rates_self, W1_rates_neigh, W1_rb_self, W1_rb_neigh,
           W2_rates_self, W2_rates_neigh, W2_rb_self, W2_rb_neigh,
           b1_rates, b1_rb, b2_rates, b2_rb):
    srcR2, dstR2 = _pad_edges(edge_index_rates[0], edge_index_rates[1])
    srcB2, dstB2 = _pad_edges(edge_index_rated_by[0], edge_index_rated_by[1])
    zeros = jnp.zeros((ZSTRIPE, H), jnp.float32)
    ones = jnp.ones((BE, H), jnp.float32)
    eye8 = jnp.eye(8, dtype=jnp.float32)
    selA = jnp.kron(eye8, jnp.eye(H, D, dtype=jnp.float32))
    selB = jnp.kron(eye8, jnp.eye(H, D, k=H, dtype=jnp.float32))

    xu_p = _pack(x_user)
    xi_p = _pack(x_item)

    degI, degU = _sc_deg(dstR2, dstB2, ones, zeros)
    degI_p, degU_p = _as_packed(degI), _as_packed(degU)

    yAr, yBr, yAb, yBb = _tcA(xu_p, xi_p,
                              _bd(W1_rates_neigh[:, :H]), _bd(W1_rates_neigh[:, H:]),
                              _bd(W1_rb_neigh[:, :H]), _bd(W1_rb_neigh[:, H:]))
    aAi, aBi, aAu, aBu = _sc_agg(_as_table(yAr), _as_table(yBr),
                                 _as_table(yAb), _as_table(yBb),
                                 srcR2, dstR2, srcB2, dstB2, zeros)
    hu, t2Ar, t2Br, hi, t2Ab, t2Bb = _tcB(
        xu_p, _as_packed(aAu), _as_packed(aBu), degU_p,
        _bd(W1_rb_self), jnp.tile(b1_rb, 8).reshape(1, 256),
        _bd(W2_rates_neigh[:, :H]), _bd(W2_rates_neigh[:, H:]),
        xi_p, _as_packed(aAi), _as_packed(aBi), degI_p,
        _bd(W1_rates_self), jnp.tile(b1_rates, 8).reshape(1, 256),
        _bd(W2_rb_neigh[:, :H]), _bd(W2_rb_neigh[:, H:]),
        selA, selB)
    a2Ai, a2Bi, a2Au, a2Bu = _sc_agg(_as_table(t2Ar), _as_table(t2Br),
                                     _as_table(t2Ab), _as_table(t2Bb),
                                     srcR2, dstR2, srcB2, dstB2, zeros)
    hu2, hi2 = _tcC(hu, _as_packed(a2Au), _as_packed(a2Bu), degU_p,
                    _bd(W2_rb_self), jnp.tile(b2_rb, 8).reshape(1, 256),
                    hi, _as_packed(a2Ai), _as_packed(a2Bi), degI_p,
                    _bd(W2_rates_self), jnp.tile(b2_rates, 8).reshape(1, 256),
                    selA, selB)
    out_u = hu2.reshape(NACC, D)[:N]
    out_i = hi2.reshape(NACC, D)[:N]
    return (out_u, out_i)


# speedup vs baseline: 1.3927x; 1.0605x over previous
import functools

import jax
import jax.numpy as jnp
from jax import lax
from jax.experimental import pallas as pl
from jax.experimental.pallas import tpu as pltpu
from jax.experimental.pallas import tpu_sc as plsc

N = 100000
D = 32
H = 16
E = 1600000
NS = 16
RPT = 784
ROWS = RPT * NS
EPAD = ROWS * 128
EPT = RPT * 128
BE = 1024
NBLK = EPT // BE
BEA = 784
NBLKA = EPT // BEA
NACC = 100352
ZSTRIPE = NACC // NS
NP = NACC // 8
BP = NP // 8
GRID = NP // BP

_mesh = plsc.VectorSubcoreMesh(core_axis_name="c", subcore_axis_name="s")


@functools.partial(
    pl.kernel,
    out_type=[jax.ShapeDtypeStruct((NACC, H), jnp.float32)] * 4,
    mesh=_mesh,
    compiler_params=pltpu.CompilerParams(use_tc_tiling_on_sc=False),
    scratch_types=[
        pltpu.VMEM_SHARED((NACC, H), jnp.float32),
        pltpu.VMEM((BEA,), jnp.int32),
        pltpu.VMEM((BEA,), jnp.int32),
        pltpu.VMEM((BEA,), jnp.int32),
        pltpu.VMEM((BEA,), jnp.int32),
        pltpu.VMEM((BEA, H), jnp.float32),
        pltpu.VMEM((BEA, H), jnp.float32),
        pltpu.SemaphoreType.DMA,
        pltpu.SemaphoreType.DMA,
        pltpu.SemaphoreType.DMA,
        pltpu.SemaphoreType.DMA,
    ],
)
def _sc_agg(tA_r, tB_r, tA_b, tB_b, srcR, dstR, srcB, dstB, zeros,
            oAi, oBi, oAu, oBu, accum,
            sidx0, sidx1, didx0, didx1, rows0, rows1,
            semg0, semg1, sems0, sems1):
    c = lax.axis_index("c")
    s = lax.axis_index("s")
    sidx = (sidx0, sidx1)
    didx = (didx0, didx1)
    rows = (rows0, rows1)
    semg = (semg0, semg1)
    sems = (sems0, sems1)

    def do_rel(table, src1, dst1, out):
        pltpu.sync_copy(zeros, accum.at[pl.ds(s * ZSTRIPE, ZSTRIPE)])
        plsc.subcore_barrier()
        base = s * EPT

        def load_idx(st, b):
            e0 = base + b * BEA
            ics = [pltpu.async_copy(src1.at[pl.ds(e0, BEA)], sidx[st], semg[st]),
                   pltpu.async_copy(dst1.at[pl.ds(e0, BEA)], didx[st], semg[st])]
            for cp in ics:
                cp.wait()

        load_idx(0, 0)
        pltpu.async_copy(table.at[sidx[0]], rows[0], semg[0])

        def pair(k, carry):
            for st in range(2):
                b = k * 2 + st
                nst = 1 - st
                pltpu.make_async_copy(table.at[sidx[st]], rows[st],
                                      semg[st]).wait()
                pltpu.async_copy(rows[st], accum.at[didx[st]], sems[st],
                                 add=True)

                @pl.when(b > 0)
                def _():
                    pltpu.make_async_copy(rows[nst], accum.at[didx[nst]],
                                          sems[nst]).wait()

                @pl.when(b + 1 < NBLKA)
                def _():
                    load_idx(nst, b + 1)
                    pltpu.async_copy(table.at[sidx[nst]], rows[nst],
                                     semg[nst])
            return carry

        lax.fori_loop(0, NBLKA // 2, pair, 0)
        last = (NBLKA - 1) % 2
        pltpu.make_async_copy(rows[last], accum.at[didx[last]],
                              sems[last]).wait()
        plsc.subcore_barrier()
        pltpu.sync_copy(accum.at[pl.ds(s * ZSTRIPE, ZSTRIPE)],
                        out.at[pl.ds(s * ZSTRIPE, ZSTRIPE)])
        plsc.subcore_barrier()

    @pl.when(c == 0)
    def _():
        do_rel(tA_r, srcR, dstR, oAi)
        do_rel(tA_b, srcB, dstB, oAu)

    @pl.when(c == 1)
    def _():
        do_rel(tB_r, srcR, dstR, oBi)
        do_rel(tB_b, srcB, dstB, oBu)


@functools.partial(
    pl.kernel,
    out_type=[jax.ShapeDtypeStruct((NACC, H), jnp.float32)] * 2,
    mesh=_mesh,
    compiler_params=pltpu.CompilerParams(use_tc_tiling_on_sc=False),
    scratch_types=[
        pltpu.VMEM_SHARED((NACC, H), jnp.float32),
        pltpu.VMEM((BE,), jnp.int32),
        pltpu.VMEM((BE, H), jnp.float32),
        pltpu.SemaphoreType.DMA,
    ],
)
def _sc_deg(dstR, dstB, ones, zeros, oI, oU, accum, didx, ones_v, sem):
    c = lax.axis_index("c")
    s = lax.axis_index("s")
    pltpu.sync_copy(ones, ones_v)

    def do_rel(dst1, out):
        pltpu.sync_copy(zeros, accum.at[pl.ds(s * ZSTRIPE, ZSTRIPE)])
        plsc.subcore_barrier()
        base = s * EPT

        def blk(b, carry):
            e0 = base + b * BE
            pltpu.sync_copy(dst1.at[pl.ds(e0, BE)], didx)
            pltpu.async_copy(ones_v, accum.at[didx], sem, add=True).wait()
            return carry

        lax.fori_loop(0, NBLK, blk, 0)
        plsc.subcore_barrier()
        pltpu.sync_copy(accum.at[pl.ds(s * ZSTRIPE, ZSTRIPE)],
                        out.at[pl.ds(s * ZSTRIPE, ZSTRIPE)])

    @pl.when(c == 0)
    def _():
        do_rel(dstR, oI)

    @pl.when(c == 1)
    def _():
        do_rel(dstB, oU)


def _rows(w):
    return pl.BlockSpec((BP, w), lambda i: (i, 0))


def _const(r, w):
    return pl.BlockSpec((r, w), lambda i: (0, 0))


_f32 = functools.partial(jnp.dot, preferred_element_type=jnp.float32)


def _tcA_body(xu, xi, wrA, wrB, wbA, wbB, yAr, yBr, yAb, yBb):
    yAr[...] = _f32(xu[...], wrA[...])
    yBr[...] = _f32(xu[...], wrB[...])
    yAb[...] = _f32(xi[...], wbA[...])
    yBb[...] = _f32(xi[...], wbB[...])


_tcA = pl.pallas_call(
    _tcA_body,
    grid=(GRID,),
    in_specs=[_rows(256), _rows(256)] + [_const(256, 128)] * 4,
    out_specs=[_rows(128)] * 4,
    out_shape=[jax.ShapeDtypeStruct((NP, 128), jnp.float32)] * 4,
)


def _mix(aA, aB, dg, eA, eB):
    inv = 1.0 / jnp.maximum(dg, 1.0)
    return _f32(aA * inv, eA) + _f32(aB * inv, eB)


def _tcB_body(xu, aAu, aBu, dgu, wsu, bu, w2rA, w2rB,
              xi, aAi, aBi, dgi, wsi, bi, w2bA, w2bB, eA, eB,
              hu, t2Ar, t2Br, hi, t2Ab, t2Bb):
    h_u = jnp.maximum(_f32(xu[...], wsu[...])
                      + _mix(aAu[...], aBu[...], dgu[...], eA[...], eB[...])
                      + bu[...], 0.0)
    hu[...] = h_u
    t2Ar[...] = _f32(h_u, w2rA[...])
    t2Br[...] = _f32(h_u, w2rB[...])
    h_i = jnp.maximum(_f32(xi[...], wsi[...])
                      + _mix(aAi[...], aBi[...], dgi[...], eA[...], eB[...])
                      + bi[...], 0.0)
    hi[...] = h_i
    t2Ab[...] = _f32(h_i, w2bA[...])
    t2Bb[...] = _f32(h_i, w2bB[...])


_tcB = pl.pallas_call(
    _tcB_body,
    grid=(GRID,),
    in_specs=[_rows(256), _rows(128), _rows(128), _rows(128),
              _const(256, 256), _const(1, 256), _const(256, 128), _const(256, 128),
              _rows(256), _rows(128), _rows(128), _rows(128),
              _const(256, 256), _const(1, 256), _const(256, 128), _const(256, 128),
              _const(128, 256), _const(128, 256)],
    out_specs=[_rows(256), _rows(128), _rows(128),
               _rows(256), _rows(128), _rows(128)],
    out_shape=[jax.ShapeDtypeStruct((NP, 256), jnp.float32),
               jax.ShapeDtypeStruct((NP, 128), jnp.float32),
               jax.ShapeDtypeStruct((NP, 128), jnp.float32),
               jax.ShapeDtypeStruct((NP, 256), jnp.float32),
               jax.ShapeDtypeStruct((NP, 128), jnp.float32),
               jax.ShapeDtypeStruct((NP, 128), jnp.float32)],
)


def _tcC_body(hu, aAu, aBu, dgu, wsu, bu,
              hi, aAi, aBi, dgi, wsi, bi, eA, eB,
              ou, oi):
    ou[...] = (_f32(hu[...], wsu[...])
               + _mix(aAu[...], aBu[...], dgu[...], eA[...], eB[...])
               + bu[...])
    oi[...] = (_f32(hi[...], wsi[...])
               + _mix(aAi[...], aBi[...], dgi[...], eA[...], eB[...])
               + bi[...])


_tcC = pl.pallas_call(
    _tcC_body,
    grid=(GRID,),
    in_specs=[_rows(256), _rows(128), _rows(128), _rows(128),
              _const(256, 256), _const(1, 256),
              _rows(256), _rows(128), _rows(128), _rows(128),
              _const(256, 256), _const(1, 256),
              _const(128, 256), _const(128, 256)],
    out_specs=[_rows(256), _rows(256)],
    out_shape=[jax.ShapeDtypeStruct((NP, 256), jnp.float32),
               jax.ShapeDtypeStruct((NP, 256), jnp.float32)],
)


def _pad_edges(src, dst):
    padn = EPAD - E
    pad_src = (jnp.arange(padn, dtype=jnp.int32) * 97) % N
    pad_dst = N + (jnp.arange(padn, dtype=jnp.int32) % (NACC - N))
    src1 = jnp.concatenate([src.astype(jnp.int32), pad_src])
    dst1 = jnp.concatenate([dst.astype(jnp.int32), pad_dst])
    return src1, dst1


def _pack(x):
    return jnp.pad(x, ((0, NACC - N), (0, 0))).reshape(NP, 8 * x.shape[1])


def _bd(W):
    return jnp.kron(jnp.eye(8, dtype=W.dtype), W)


def _as_table(p):
    return p.reshape(NACC, H)


def _as_packed(t):
    return t.reshape(NP, 128)


def kernel(x_user, x_item, edge_index_rates, edge_index_rated_by,
           W1_rates_self, W1_rates_neigh, W1_rb_self, W1_rb_neigh,
           W2_rates_self, W2_rates_neigh, W2_rb_self, W2_rb_neigh,
           b1_rates, b1_rb, b2_rates, b2_rb):
    srcR2, dstR2 = _pad_edges(edge_index_rates[0], edge_index_rates[1])
    srcB2, dstB2 = _pad_edges(edge_index_rated_by[0], edge_index_rated_by[1])
    zeros = jnp.zeros((ZSTRIPE, H), jnp.float32)
    ones = jnp.ones((BE, H), jnp.float32)
    eye8 = jnp.eye(8, dtype=jnp.float32)
    selA = jnp.kron(eye8, jnp.eye(H, D, dtype=jnp.float32))
    selB = jnp.kron(eye8, jnp.eye(H, D, k=H, dtype=jnp.float32))

    xu_p = _pack(x_user)
    xi_p = _pack(x_item)

    degI, degU = _sc_deg(dstR2, dstB2, ones, zeros)
    degI_p, degU_p = _as_packed(degI), _as_packed(degU)

    yAr, yBr, yAb, yBb = _tcA(xu_p, xi_p,
                              _bd(W1_rates_neigh[:, :H]), _bd(W1_rates_neigh[:, H:]),
                              _bd(W1_rb_neigh[:, :H]), _bd(W1_rb_neigh[:, H:]))
    aAi, aBi, aAu, aBu = _sc_agg(_as_table(yAr), _as_table(yBr),
                                 _as_table(yAb), _as_table(yBb),
                                 srcR2, dstR2, srcB2, dstB2, zeros)
    hu, t2Ar, t2Br, hi, t2Ab, t2Bb = _tcB(
        xu_p, _as_packed(aAu), _as_packed(aBu), degU_p,
        _bd(W1_rb_self), jnp.tile(b1_rb, 8).reshape(1, 256),
        _bd(W2_rates_neigh[:, :H]), _bd(W2_rates_neigh[:, H:]),
        xi_p, _as_packed(aAi), _as_packed(aBi), degI_p,
        _bd(W1_rates_self), jnp.tile(b1_rates, 8).reshape(1, 256),
        _bd(W2_rb_neigh[:, :H]), _bd(W2_rb_neigh[:, H:]),
        selA, selB)
    a2Ai, a2Bi, a2Au, a2Bu = _sc_agg(_as_table(t2Ar), _as_table(t2Br),
                                     _as_table(t2Ab), _as_table(t2Bb),
                                     srcR2, dstR2, srcB2, dstB2, zeros)
    hu2, hi2 = _tcC(hu, _as_packed(a2Au), _as_packed(a2Bu), degU_p,
                    _bd(W2_rb_self), jnp.tile(b2_rb, 8).reshape(1, 256),
                    hi, _as_packed(a2Ai), _as_packed(a2Bi), degI_p,
                    _bd(W2_rates_self), jnp.tile(b2_rates, 8).reshape(1, 256),
                    selA, selB)
    out_u = hu2.reshape(NACC, D)[:N]
    out_i = hi2.reshape(NACC, D)[:N]
    return (out_u, out_i)


# speedup vs baseline: 1.5046x; 1.0803x over previous
import functools

import jax
import jax.numpy as jnp
from jax import lax
from jax.experimental import pallas as pl
from jax.experimental.pallas import tpu as pltpu
from jax.experimental.pallas import tpu_sc as plsc

N = 100000
D = 32
H = 16
E = 1600000
NS = 16
RPT = 784
ROWS = RPT * NS
EPAD = ROWS * 128
EPT = RPT * 128
BE = 1024
NBLK = EPT // BE
BEA = 512
NBLKA = EPT // BEA
NACC = 100352
ZSTRIPE = NACC // NS
NP = NACC // 8
BP = NP // 8
GRID = NP // BP

_mesh = plsc.VectorSubcoreMesh(core_axis_name="c", subcore_axis_name="s")


@functools.partial(
    pl.kernel,
    out_type=[jax.ShapeDtypeStruct((NACC, H), jnp.float32)] * 4,
    mesh=_mesh,
    compiler_params=pltpu.CompilerParams(use_tc_tiling_on_sc=False),
    scratch_types=[
        pltpu.VMEM_SHARED((NACC, H), jnp.float32),
        [pltpu.VMEM((BEA,), jnp.int32)] * 4,
        [pltpu.VMEM((BEA,), jnp.int32)] * 4,
        [pltpu.VMEM((BEA, H), jnp.float32)] * 2,
        [pltpu.SemaphoreType.DMA] * 2,
        [pltpu.SemaphoreType.DMA] * 2,
        [pltpu.SemaphoreType.DMA] * 4,
    ],
)
def _sc_agg(tA_r, tB_r, tA_b, tB_b, srcR, dstR, srcB, dstB, zeros,
            oAi, oBi, oAu, oBu, accum, sidx, didx, rows, semg, sems, semi):
    c = lax.axis_index("c")
    s = lax.axis_index("s")

    def do_rel(table, src1, dst1, out):
        pltpu.sync_copy(zeros, accum.at[pl.ds(s * ZSTRIPE, ZSTRIPE)])
        plsc.subcore_barrier()
        base = s * EPT

        def fire_idx(q, b):
            e0 = base + b * BEA
            pltpu.async_copy(src1.at[pl.ds(e0, BEA)], sidx[q], semi[q])
            pltpu.async_copy(dst1.at[pl.ds(e0, BEA)], didx[q], semi[q])

        def drain_idx(q, b):
            e0 = base + b * BEA
            pltpu.make_async_copy(src1.at[pl.ds(e0, BEA)], sidx[q],
                                  semi[q]).wait()
            pltpu.make_async_copy(dst1.at[pl.ds(e0, BEA)], didx[q],
                                  semi[q]).wait()

        fire_idx(0, 0)
        fire_idx(1, 1)
        drain_idx(0, 0)
        pltpu.async_copy(table.at[sidx[0]], rows[0], semg[0])

        def quad(k, carry):
            for st4 in range(4):
                b = k * 4 + st4
                st = st4 % 2
                nst = 1 - st

                @pl.when(b + 2 < NBLKA)
                def _():
                    fire_idx((st4 + 2) % 4, b + 2)

                pltpu.make_async_copy(table.at[sidx[st4]], rows[st],
                                      semg[st]).wait()
                pltpu.async_copy(rows[st], accum.at[didx[st4]], sems[st],
                                 add=True)

                @pl.when(b > 0)
                def _():
                    pltpu.make_async_copy(rows[nst],
                                          accum.at[didx[(st4 + 3) % 4]],
                                          sems[nst]).wait()

                @pl.when(b + 1 < NBLKA)
                def _():
                    drain_idx((st4 + 1) % 4, b + 1)
                    pltpu.async_copy(table.at[sidx[(st4 + 1) % 4]],
                                     rows[nst], semg[nst])
            return carry

        lax.fori_loop(0, NBLKA // 4, quad, 0)
        pltpu.make_async_copy(rows[(NBLKA - 1) % 2],
                              accum.at[didx[(NBLKA - 1) % 4]],
                              sems[(NBLKA - 1) % 2]).wait()
        plsc.subcore_barrier()
        pltpu.sync_copy(accum.at[pl.ds(s * ZSTRIPE, ZSTRIPE)],
                        out.at[pl.ds(s * ZSTRIPE, ZSTRIPE)])
        plsc.subcore_barrier()

    @pl.when(c == 0)
    def _():
        do_rel(tA_r, srcR, dstR, oAi)
        do_rel(tA_b, srcB, dstB, oAu)

    @pl.when(c == 1)
    def _():
        do_rel(tB_r, srcR, dstR, oBi)
        do_rel(tB_b, srcB, dstB, oBu)


@functools.partial(
    pl.kernel,
    out_type=[jax.ShapeDtypeStruct((NACC, H), jnp.float32)] * 2,
    mesh=_mesh,
    compiler_params=pltpu.CompilerParams(use_tc_tiling_on_sc=False),
    scratch_types=[
        pltpu.VMEM_SHARED((NACC, H), jnp.float32),
        pltpu.VMEM((BE,), jnp.int32),
        pltpu.VMEM((BE, H), jnp.float32),
        pltpu.SemaphoreType.DMA,
    ],
)
def _sc_deg(dstR, dstB, ones, zeros, oI, oU, accum, didx, ones_v, sem):
    c = lax.axis_index("c")
    s = lax.axis_index("s")
    pltpu.sync_copy(ones, ones_v)

    def do_rel(dst1, out):
        pltpu.sync_copy(zeros, accum.at[pl.ds(s * ZSTRIPE, ZSTRIPE)])
        plsc.subcore_barrier()
        base = s * EPT

        def blk(b, carry):
            e0 = base + b * BE
            pltpu.sync_copy(dst1.at[pl.ds(e0, BE)], didx)
            pltpu.async_copy(ones_v, accum.at[didx], sem, add=True).wait()
            return carry

        lax.fori_loop(0, NBLK, blk, 0)
        plsc.subcore_barrier()
        pltpu.sync_copy(accum.at[pl.ds(s * ZSTRIPE, ZSTRIPE)],
                        out.at[pl.ds(s * ZSTRIPE, ZSTRIPE)])

    @pl.when(c == 0)
    def _():
        do_rel(dstR, oI)

    @pl.when(c == 1)
    def _():
        do_rel(dstB, oU)


def _rows(w):
    return pl.BlockSpec((BP, w), lambda i: (i, 0))


def _const(r, w):
    return pl.BlockSpec((r, w), lambda i: (0, 0))


_f32 = functools.partial(jnp.dot, preferred_element_type=jnp.float32)


def _tcA_body(xu, xi, wrA, wrB, wbA, wbB, yAr, yBr, yAb, yBb):
    yAr[...] = _f32(xu[...], wrA[...])
    yBr[...] = _f32(xu[...], wrB[...])
    yAb[...] = _f32(xi[...], wbA[...])
    yBb[...] = _f32(xi[...], wbB[...])


_tcA = pl.pallas_call(
    _tcA_body,
    grid=(GRID,),
    in_specs=[_rows(256), _rows(256)] + [_const(256, 128)] * 4,
    out_specs=[_rows(128)] * 4,
    out_shape=[jax.ShapeDtypeStruct((NP, 128), jnp.float32)] * 4,
)


def _mix(aA, aB, dg, eA, eB):
    inv = 1.0 / jnp.maximum(dg, 1.0)
    return _f32(aA * inv, eA) + _f32(aB * inv, eB)


def _tcB_body(xu, aAu, aBu, dgu, wsu, bu, w2rA, w2rB,
              xi, aAi, aBi, dgi, wsi, bi, w2bA, w2bB, eA, eB,
              hu, t2Ar, t2Br, hi, t2Ab, t2Bb):
    h_u = jnp.maximum(_f32(xu[...], wsu[...])
                      + _mix(aAu[...], aBu[...], dgu[...], eA[...], eB[...])
                      + bu[...], 0.0)
    hu[...] = h_u
    t2Ar[...] = _f32(h_u, w2rA[...])
    t2Br[...] = _f32(h_u, w2rB[...])
    h_i = jnp.maximum(_f32(xi[...], wsi[...])
                      + _mix(aAi[...], aBi[...], dgi[...], eA[...], eB[...])
                      + bi[...], 0.0)
    hi[...] = h_i
    t2Ab[...] = _f32(h_i, w2bA[...])
    t2Bb[...] = _f32(h_i, w2bB[...])


_tcB = pl.pallas_call(
    _tcB_body,
    grid=(GRID,),
    in_specs=[_rows(256), _rows(128), _rows(128), _rows(128),
              _const(256, 256), _const(1, 256), _const(256, 128), _const(256, 128),
              _rows(256), _rows(128), _rows(128), _rows(128),
              _const(256, 256), _const(1, 256), _const(256, 128), _const(256, 128),
              _const(128, 256), _const(128, 256)],
    out_specs=[_rows(256), _rows(128), _rows(128),
               _rows(256), _rows(128), _rows(128)],
    out_shape=[jax.ShapeDtypeStruct((NP, 256), jnp.float32),
               jax.ShapeDtypeStruct((NP, 128), jnp.float32),
               jax.ShapeDtypeStruct((NP, 128), jnp.float32),
               jax.ShapeDtypeStruct((NP, 256), jnp.float32),
               jax.ShapeDtypeStruct((NP, 128), jnp.float32),
               jax.ShapeDtypeStruct((NP, 128), jnp.float32)],
)


def _tcC_body(hu, aAu, aBu, dgu, wsu, bu,
              hi, aAi, aBi, dgi, wsi, bi, eA, eB,
              ou, oi):
    ou[...] = (_f32(hu[...], wsu[...])
               + _mix(aAu[...], aBu[...], dgu[...], eA[...], eB[...])
               + bu[...])
    oi[...] = (_f32(hi[...], wsi[...])
               + _mix(aAi[...], aBi[...], dgi[...], eA[...], eB[...])
               + bi[...])


_tcC = pl.pallas_call(
    _tcC_body,
    grid=(GRID,),
    in_specs=[_rows(256), _rows(128), _rows(128), _rows(128),
              _const(256, 256), _const(1, 256),
              _rows(256), _rows(128), _rows(128), _rows(128),
              _const(256, 256), _const(1, 256),
              _const(128, 256), _const(128, 256)],
    out_specs=[_rows(256), _rows(256)],
    out_shape=[jax.ShapeDtypeStruct((NP, 256), jnp.float32),
               jax.ShapeDtypeStruct((NP, 256), jnp.float32)],
)


def _pad_edges(src, dst):
    padn = EPAD - E
    pad_src = (jnp.arange(padn, dtype=jnp.int32) * 97) % N
    pad_dst = N + (jnp.arange(padn, dtype=jnp.int32) % (NACC - N))
    src1 = jnp.concatenate([src.astype(jnp.int32), pad_src])
    dst1 = jnp.concatenate([dst.astype(jnp.int32), pad_dst])
    return src1, dst1


def _pack(x):
    return jnp.pad(x, ((0, NACC - N), (0, 0))).reshape(NP, 8 * x.shape[1])


def _bd(W):
    return jnp.kron(jnp.eye(8, dtype=W.dtype), W)


def _as_table(p):
    return p.reshape(NACC, H)


def _as_packed(t):
    return t.reshape(NP, 128)


def kernel(x_user, x_item, edge_index_rates, edge_index_rated_by,
           W1_rates_self, W1_rates_neigh, W1_rb_self, W1_rb_neigh,
           W2_rates_self, W2_rates_neigh, W2_rb_self, W2_rb_neigh,
           b1_rates, b1_rb, b2_rates, b2_rb):
    srcR2, dstR2 = _pad_edges(edge_index_rates[0], edge_index_rates[1])
    srcB2, dstB2 = _pad_edges(edge_index_rated_by[0], edge_index_rated_by[1])
    zeros = jnp.zeros((ZSTRIPE, H), jnp.float32)
    ones = jnp.ones((BE, H), jnp.float32)
    eye8 = jnp.eye(8, dtype=jnp.float32)
    selA = jnp.kron(eye8, jnp.eye(H, D, dtype=jnp.float32))
    selB = jnp.kron(eye8, jnp.eye(H, D, k=H, dtype=jnp.float32))

    xu_p = _pack(x_user)
    xi_p = _pack(x_item)

    degI, degU = _sc_deg(dstR2, dstB2, ones, zeros)
    degI_p, degU_p = _as_packed(degI), _as_packed(degU)

    yAr, yBr, yAb, yBb = _tcA(xu_p, xi_p,
                              _bd(W1_rates_neigh[:, :H]), _bd(W1_rates_neigh[:, H:]),
                              _bd(W1_rb_neigh[:, :H]), _bd(W1_rb_neigh[:, H:]))
    aAi, aBi, aAu, aBu = _sc_agg(_as_table(yAr), _as_table(yBr),
                                 _as_table(yAb), _as_table(yBb),
                                 srcR2, dstR2, srcB2, dstB2, zeros)
    hu, t2Ar, t2Br, hi, t2Ab, t2Bb = _tcB(
        xu_p, _as_packed(aAu), _as_packed(aBu), degU_p,
        _bd(W1_rb_self), jnp.tile(b1_rb, 8).reshape(1, 256),
        _bd(W2_rates_neigh[:, :H]), _bd(W2_rates_neigh[:, H:]),
        xi_p, _as_packed(aAi), _as_packed(aBi), degI_p,
        _bd(W1_rates_self), jnp.tile(b1_rates, 8).reshape(1, 256),
        _bd(W2_rb_neigh[:, :H]), _bd(W2_rb_neigh[:, H:]),
        selA, selB)
    a2Ai, a2Bi, a2Au, a2Bu = _sc_agg(_as_table(t2Ar), _as_table(t2Br),
                                     _as_table(t2Ab), _as_table(t2Bb),
                                     srcR2, dstR2, srcB2, dstB2, zeros)
    hu2, hi2 = _tcC(hu, _as_packed(a2Au), _as_packed(a2Bu), degU_p,
                    _bd(W2_rb_self), jnp.tile(b2_rb, 8).reshape(1, 256),
                    hi, _as_packed(a2Ai), _as_packed(a2Bi), degI_p,
                    _bd(W2_rates_self), jnp.tile(b2_rates, 8).reshape(1, 256),
                    selA, selB)
    out_u = hu2.reshape(NACC, D)[:N]
    out_i = hi2.reshape(NACC, D)[:N]
    return (out_u, out_i)
